# baseline (speedup 1.0000x reference)
import functools

import jax
import jax.numpy as jnp
from jax import lax
from jax.experimental import pallas as pl
from jax.experimental.pallas import tpu as pltpu
from jax.experimental.pallas import tpu_sc as plsc

N = 10000
E = 320000
D = 128

NC = 2
NS = 16
NW = NC * NS

EPW0 = E // NW
K = 112
PAD = 80
EPW = EPW0 + PAD
CH = EPW // K
NP = 10240
RPT = NP // NS


def _sc_aggregate(h, edata):
    mesh = plsc.VectorSubcoreMesh(core_axis_name="c", subcore_axis_name="s",
                                  num_cores=NC)

    @functools.partial(
        pl.kernel, mesh=mesh,
        out_type=jax.ShapeDtypeStruct((NC, NP, D), jnp.float32),
        scratch_types=[
            pltpu.VMEM((3, K), jnp.int32),
            pltpu.VMEM((3, K), jnp.int32),
            pltpu.VMEM((3, K), jnp.int32),
            pltpu.VMEM((K, D), jnp.float32),
            pltpu.VMEM((K, D), jnp.float32),
            pltpu.VMEM((K, D), jnp.float32),
            pltpu.VMEM_SHARED((NP, D), jnp.float32),
            pltpu.SemaphoreType.DMA,
            pltpu.SemaphoreType.DMA,
            pltpu.SemaphoreType.DMA,
            pltpu.SemaphoreType.DMA,
            pltpu.SemaphoreType.DMA,
            pltpu.SemaphoreType.DMA,
            pltpu.SemaphoreType.DMA,
            pltpu.SemaphoreType.DMA,
            pltpu.SemaphoreType.DMA,
        ],
    )
    def agg_kernel(h_hbm, ed_hbm, out_hbm,
                   eb0, eb1, eb2, rows0, rows1, rows2, agg_sh,
                   esem0, esem1, esem2, gsem0, gsem1, gsem2,
                   csem0, csem1, csem2):
        c = lax.axis_index("c")
        s = lax.axis_index("s")
        wid = s * NC + c
        ch0 = wid * CH

        bufs = ((eb0, esem0, rows0, gsem0, csem0),
                (eb1, esem1, rows1, gsem1, csem1),
                (eb2, esem2, rows2, gsem2, csem2))

        def load_edges(i, r):
            eb, esem = bufs[r][0], bufs[r][1]
            pltpu.async_copy(ed_hbm.at[ch0 + i], eb, esem)

        def wait_edges(i, r):
            eb, esem = bufs[r][0], bufs[r][1]
            pltpu.make_async_copy(ed_hbm.at[ch0 + i], eb, esem).wait()

        def start_gather(r):
            eb, _, rows, gsem, _ = bufs[r]
            pltpu.async_copy(h_hbm.at[eb.at[0]], rows, gsem)

        def wait_gather(r):
            eb, _, rows, gsem, _ = bufs[r]
            pltpu.make_async_copy(h_hbm.at[eb.at[0]], rows, gsem).wait()

        def start_scatter(r):
            eb, _, rows, _, csem = bufs[r]
            pltpu.async_copy(rows, agg_sh.at[eb.at[1]], csem, add=True)

        def wait_scatter(r):
            eb, _, rows, _, csem = bufs[r]
            pltpu.make_async_copy(rows, agg_sh.at[eb.at[1]], csem).wait()

        def scale(r):
            eb, _, rows, _, _ = bufs[r]

            def edge16(g, _):
                nv = lax.bitcast_convert_type(
                    eb[2, pl.ds(g * 16, 16)], jnp.float32)
                for l in range(16):
                    k = g * 16 + l
                    nk = nv[l]
                    for j in range(D // 16):
                        sl = pl.ds(j * 16, 16)
                        rows[k, sl] = rows[k, sl] * nk
                return 0

            lax.fori_loop(0, K // 16, edge16, 0, unroll=2)

        load_edges(0, 0)
        load_edges(1, 1)
        load_edges(2, 2)

        zero16 = jnp.zeros((16,), jnp.float32)

        def zrow(rr, _):
            for j in range(D // 16):
                rows0[rr, pl.ds(j * 16, 16)] = zero16
            return 0

        lax.fori_loop(0, 80, zrow, 0)

        def zcopy(t, _):
            pltpu.sync_copy(rows0.at[pl.ds(0, 80)],
                            agg_sh.at[pl.ds(s * RPT + t * 80, 80)])
            return 0

        lax.fori_loop(0, RPT // 80, zcopy, 0)
        plsc.subcore_barrier()

        wait_edges(0, 0)
        start_gather(0)

        for i in range(3):
            r, rn, rn2 = i % 3, (i + 1) % 3, (i + 2) % 3
            wait_edges(i + 1, rn)
            start_gather(rn)
            wait_gather(r)
            scale(r)
            start_scatter(r)
            if i >= 1:
                wait_scatter(rn2)
                load_edges(i + 2, rn2)

        def triple(t, _):
            for r in range(3):
                i = 3 * t + r
                rn, rn2 = (r + 1) % 3, (r + 2) % 3
                wait_edges(i + 1, rn)
                start_gather(rn)
                wait_gather(r)
                scale(r)
                start_scatter(r)
                wait_scatter(rn2)
                load_edges(i + 2, rn2)
            return 0

        lax.fori_loop(1, CH // 3 - 1, triple, 0)

        for i in range(CH - 3, CH):
            r, rn, rn2 = i % 3, (i + 1) % 3, (i + 2) % 3
            if i + 1 < CH:
                wait_edges(i + 1, rn)
                start_gather(rn)
            wait_gather(r)
            scale(r)
            start_scatter(r)
            if i + 2 < CH:
                wait_scatter(rn2)
                load_edges(i + 2, rn2)
            elif i + 1 < CH:
                wait_scatter(rn2)

        wait_scatter((CH - 2) % 3)
        wait_scatter((CH - 1) % 3)
        plsc.subcore_barrier()

        pltpu.sync_copy(agg_sh.at[pl.ds(s * RPT, RPT)],
                        out_hbm.at[c, pl.ds(s * RPT, RPT)])

    return agg_kernel(h, edata)


_R = 1000


def _tc_in_kernel(x_ref, w_ref, b_ref, o_ref):
    z = jnp.dot(x_ref[...], w_ref[...], preferred_element_type=jnp.float32)
    o_ref[...] = jnp.maximum(z + b_ref[...], 0.0)


def _tc_in(x, W0, b0):
    return pl.pallas_call(
        _tc_in_kernel,
        grid=(N // _R,),
        in_specs=[
            pl.BlockSpec((_R, D), lambda i: (i, 0)),
            pl.BlockSpec((D, D), lambda i: (0, 0)),
            pl.BlockSpec((1, D), lambda i: (0, 0)),
        ],
        out_specs=pl.BlockSpec((_R, D), lambda i: (i, 0)),
        out_shape=jax.ShapeDtypeStruct((N, D), jnp.float32),
    )(x, W0, b0.reshape(1, D))


def _tc_layer_kernel(p_ref, w_ref, b_ref, x0_ref, o_ref):
    a = p_ref[0] + p_ref[1]
    z = jnp.dot(a, w_ref[...], preferred_element_type=jnp.float32)
    o_ref[...] = jnp.maximum(z + b_ref[...] + x0_ref[...], 0.0)


def _tc_layer(parts, W, b, x0):
    return pl.pallas_call(
        _tc_layer_kernel,
        grid=(N // _R,),
        in_specs=[
            pl.BlockSpec((NC, _R, D), lambda i: (0, i, 0)),
            pl.BlockSpec((D, D), lambda i: (0, 0)),
            pl.BlockSpec((1, D), lambda i: (0, 0)),
            pl.BlockSpec((_R, D), lambda i: (i, 0)),
        ],
        out_specs=pl.BlockSpec((_R, D), lambda i: (i, 0)),
        out_shape=jax.ShapeDtypeStruct((N, D), jnp.float32),
    )(parts, W, b.reshape(1, D), x0)


def _tc_last_kernel(p_ref, w_ref, b_ref, x0_ref, wo_ref, bo_ref, o_ref):
    a = p_ref[0] + p_ref[1]
    hh = jnp.dot(a, w_ref[...], preferred_element_type=jnp.float32)
    hh = jnp.maximum(hh + b_ref[...] + x0_ref[...], 0.0)
    z = jnp.dot(hh, wo_ref[...], preferred_element_type=jnp.float32)
    z = z + bo_ref[...]
    m = jnp.max(z, axis=1, keepdims=True)
    lse = jnp.log(jnp.sum(jnp.exp(z - m), axis=1, keepdims=True))
    o_ref[...] = z - m - lse


def _tc_last(parts, W, b, x0, Wout, bout):
    dout = Wout.shape[1]
    return pl.pallas_call(
        _tc_last_kernel,
        grid=(N // _R,),
        in_specs=[
            pl.BlockSpec((NC, _R, D), lambda i: (0, i, 0)),
            pl.BlockSpec((D, D), lambda i: (0, 0)),
            pl.BlockSpec((1, D), lambda i: (0, 0)),
            pl.BlockSpec((_R, D), lambda i: (i, 0)),
            pl.BlockSpec((D, dout), lambda i: (0, 0)),
            pl.BlockSpec((1, dout), lambda i: (0, 0)),
        ],
        out_specs=pl.BlockSpec((_R, dout), lambda i: (i, 0)),
        out_shape=jax.ShapeDtypeStruct((N, dout), jnp.float32),
    )(parts, W, b.reshape(1, D), x0, Wout, bout.reshape(1, dout))


def kernel(x, edge_index, norm, W0, b0, Wc, bc, Wout, bout):
    spread = (jnp.arange(PAD, dtype=jnp.int32) * 911) % N
    pad_blk = jnp.broadcast_to(spread, (NW, PAD))
    src = jnp.concatenate(
        [edge_index[0].reshape(NW, EPW0), pad_blk], axis=1)
    dst = jnp.concatenate(
        [edge_index[1].reshape(NW, EPW0), pad_blk], axis=1)
    nrm = jnp.concatenate(
        [lax.bitcast_convert_type(norm, jnp.int32).reshape(NW, EPW0),
         jnp.zeros((NW, PAD), jnp.int32)], axis=1)
    edata = jnp.stack([src.reshape(NW, CH, K),
                       dst.reshape(NW, CH, K),
                       nrm.reshape(NW, CH, K)], axis=2).reshape(NW * CH, 3, K)

    h = _tc_in(x, W0, b0)
    x0 = h
    L = Wc.shape[0]
    for i in range(L - 1):
        parts = _sc_aggregate(h, edata)
        h = _tc_layer(parts, Wc[i], bc[i], x0)
    parts = _sc_aggregate(h, edata)
    return _tc_last(parts, Wc[L - 1], bc[L - 1], x0, Wout, bout)


# speedup vs baseline: 1.0264x; 1.0264x over previous
import functools

import jax
import jax.numpy as jnp
from jax import lax
from jax.experimental import pallas as pl
from jax.experimental.pallas import tpu as pltpu
from jax.experimental.pallas import tpu_sc as plsc

N = 10000
E = 320000
D = 128

NC = 2
NS = 16
NW = NC * NS

EPW0 = E // NW
K = 112
PAD = 80
EPW = EPW0 + PAD
CH = EPW // K
NP = 10240
RPT = NP // NS


def _sc_aggregate(h, src, dst, norm):
    mesh = plsc.VectorSubcoreMesh(core_axis_name="c", subcore_axis_name="s",
                                  num_cores=NC)

    @functools.partial(
        pl.kernel, mesh=mesh,
        out_type=jax.ShapeDtypeStruct((NC, NP, D), jnp.float32),
        scratch_types=[
            pltpu.VMEM((K,), jnp.int32),
            pltpu.VMEM((K,), jnp.int32),
            pltpu.VMEM((K,), jnp.int32),
            pltpu.VMEM((K,), jnp.int32),
            pltpu.VMEM((K,), jnp.int32),
            pltpu.VMEM((K,), jnp.int32),
            pltpu.VMEM((K,), jnp.float32),
            pltpu.VMEM((K,), jnp.float32),
            pltpu.VMEM((K,), jnp.float32),
            pltpu.VMEM((K, D), jnp.float32),
            pltpu.VMEM((K, D), jnp.float32),
            pltpu.VMEM((K, D), jnp.float32),
            pltpu.VMEM_SHARED((NP, D), jnp.float32),
            pltpu.SemaphoreType.DMA,
            pltpu.SemaphoreType.DMA,
            pltpu.SemaphoreType.DMA,
            pltpu.SemaphoreType.DMA,
            pltpu.SemaphoreType.DMA,
            pltpu.SemaphoreType.DMA,
            pltpu.SemaphoreType.DMA,
            pltpu.SemaphoreType.DMA,
            pltpu.SemaphoreType.DMA,
            pltpu.SemaphoreType.DMA,
            pltpu.SemaphoreType.DMA,
            pltpu.SemaphoreType.DMA,
            pltpu.SemaphoreType.DMA,
            pltpu.SemaphoreType.DMA,
            pltpu.SemaphoreType.DMA,
        ],
    )
    def agg_kernel(h_hbm, src_hbm, dst_hbm, norm_hbm, out_hbm,
                   src0, src1, src2, dst0, dst1, dst2,
                   norm0, norm1, norm2, rows0, rows1, rows2, agg_sh,
                   ssem0, ssem1, ssem2, dsem0, dsem1, dsem2,
                   nsem0, nsem1, nsem2,
                   gsem0, gsem1, gsem2, csem0, csem1, csem2):
        c = lax.axis_index("c")
        s = lax.axis_index("s")
        wid = s * NC + c
        e0 = wid * EPW

        pltpu.async_copy(src_hbm.at[pl.ds(e0, K)], src0, ssem0)
        pltpu.async_copy(src_hbm.at[pl.ds(e0 + K, K)], src1, ssem1)
        pltpu.async_copy(src_hbm.at[pl.ds(e0 + 2 * K, K)], src2, ssem2)
        pltpu.async_copy(dst_hbm.at[pl.ds(e0, K)], dst0, dsem0)
        pltpu.async_copy(dst_hbm.at[pl.ds(e0 + K, K)], dst1, dsem1)
        pltpu.async_copy(dst_hbm.at[pl.ds(e0 + 2 * K, K)], dst2, dsem2)
        pltpu.async_copy(norm_hbm.at[pl.ds(e0, K)], norm0, nsem0)
        pltpu.async_copy(norm_hbm.at[pl.ds(e0 + K, K)], norm1, nsem1)
        pltpu.async_copy(norm_hbm.at[pl.ds(e0 + 2 * K, K)], norm2, nsem2)

        zero16 = jnp.zeros((16,), jnp.float32)

        def zrow(r, _):
            for j in range(D // 16):
                rows0[r, pl.ds(j * 16, 16)] = zero16
            return 0

        lax.fori_loop(0, 80, zrow, 0)

        def zcopy(t, _):
            pltpu.sync_copy(rows0.at[pl.ds(0, 80)],
                            agg_sh.at[pl.ds(s * RPT + t * 80, 80)])
            return 0

        lax.fori_loop(0, RPT // 80, zcopy, 0)
        plsc.subcore_barrier()

        pltpu.make_async_copy(src_hbm.at[pl.ds(e0, K)], src0, ssem0).wait()
        pltpu.async_copy(h_hbm.at[src0], rows0, gsem0)

        bufs = ((src0, ssem0, dst0, dsem0, norm0, nsem0, rows0, gsem0, csem0),
                (src1, ssem1, dst1, dsem1, norm1, nsem1, rows1, gsem1, csem1),
                (src2, ssem2, dst2, dsem2, norm2, nsem2, rows2, gsem2, csem2))

        def prefetch_gather(i, r):
            src_b, ssem_b, rows_b, gsem_b = (
                bufs[r][0], bufs[r][1], bufs[r][6], bufs[r][7])
            pltpu.make_async_copy(
                src_hbm.at[pl.ds(e0 + i * K, K)], src_b, ssem_b).wait()
            pltpu.async_copy(h_hbm.at[src_b], rows_b, gsem_b)

        def process(i, r):
            (src_c, ssem_c, dst_c, dsem_c, norm_c, nsem_c,
             rows_c, gsem_c, csem_c) = bufs[r]
            pltpu.make_async_copy(
                norm_hbm.at[pl.ds(e0 + i * K, K)], norm_c, nsem_c).wait()
            pltpu.make_async_copy(h_hbm.at[src_c], rows_c, gsem_c).wait()

            def edge16(g, _):
                nv = norm_c[pl.ds(g * 16, 16)]
                for l in range(16):
                    k = g * 16 + l
                    nk = nv[l]
                    for j in range(D // 16):
                        sl = pl.ds(j * 16, 16)
                        rows_c[k, sl] = rows_c[k, sl] * nk
                return 0

            lax.fori_loop(0, K // 16, edge16, 0, unroll=2)
            pltpu.make_async_copy(
                dst_hbm.at[pl.ds(e0 + i * K, K)], dst_c, dsem_c).wait()
            pltpu.async_copy(rows_c, agg_sh.at[dst_c], csem_c, add=True)

        def retire_scatter(r):
            dst_b, rows_b, csem_b = bufs[r][2], bufs[r][6], bufs[r][8]
            pltpu.make_async_copy(rows_b, agg_sh.at[dst_b], csem_b).wait()

        def refill(i, r):
            (src_b, ssem_b, dst_b, dsem_b, norm_b, nsem_b) = bufs[r][:6]
            pltpu.async_copy(
                src_hbm.at[pl.ds(e0 + i * K, K)], src_b, ssem_b)
            pltpu.async_copy(
                dst_hbm.at[pl.ds(e0 + i * K, K)], dst_b, dsem_b)
            pltpu.async_copy(
                norm_hbm.at[pl.ds(e0 + i * K, K)], norm_b, nsem_b)

        for i in range(3):
            r, rn, rn2 = i % 3, (i + 1) % 3, (i + 2) % 3
            prefetch_gather(i + 1, rn)
            process(i, r)
            if i >= 1:
                retire_scatter(rn2)
                refill(i + 2, rn2)

        def triple(t, _):
            for r in range(3):
                i = 3 * t + r
                prefetch_gather(i + 1, (r + 1) % 3)
                process(i, r)
                retire_scatter((r + 2) % 3)
                refill(i + 2, (r + 2) % 3)
            return 0

        lax.fori_loop(1, CH // 3 - 1, triple, 0)

        for i in range(CH - 3, CH):
            r, rn, rn2 = i % 3, (i + 1) % 3, (i + 2) % 3
            if i + 1 < CH:
                prefetch_gather(i + 1, rn)
            process(i, r)
            if i + 2 < CH:
                retire_scatter(rn2)
                refill(i + 2, rn2)
            elif i + 1 < CH:
                retire_scatter(rn2)

        retire_scatter((CH - 2) % 3)
        retire_scatter((CH - 1) % 3)
        plsc.subcore_barrier()

        pltpu.sync_copy(agg_sh.at[pl.ds(s * RPT, RPT)],
                        out_hbm.at[c, pl.ds(s * RPT, RPT)])

    return agg_kernel(h, src, dst, norm)


_R = 1000


def _tc_in_kernel(x_ref, w_ref, b_ref, o_ref):
    z = jnp.dot(x_ref[...], w_ref[...], preferred_element_type=jnp.float32)
    o_ref[...] = jnp.maximum(z + b_ref[...], 0.0)


def _tc_in(x, W0, b0):
    return pl.pallas_call(
        _tc_in_kernel,
        grid=(N // _R,),
        in_specs=[
            pl.BlockSpec((_R, D), lambda i: (i, 0)),
            pl.BlockSpec((D, D), lambda i: (0, 0)),
            pl.BlockSpec((1, D), lambda i: (0, 0)),
        ],
        out_specs=pl.BlockSpec((_R, D), lambda i: (i, 0)),
        out_shape=jax.ShapeDtypeStruct((N, D), jnp.float32),
    )(x, W0, b0.reshape(1, D))


def _tc_layer_kernel(p_ref, w_ref, b_ref, x0_ref, o_ref):
    a = p_ref[0] + p_ref[1]
    z = jnp.dot(a, w_ref[...], preferred_element_type=jnp.float32)
    o_ref[...] = jnp.maximum(z + b_ref[...] + x0_ref[...], 0.0)


def _tc_layer(parts, W, b, x0):
    return pl.pallas_call(
        _tc_layer_kernel,
        grid=(N // _R,),
        in_specs=[
            pl.BlockSpec((NC, _R, D), lambda i: (0, i, 0)),
            pl.BlockSpec((D, D), lambda i: (0, 0)),
            pl.BlockSpec((1, D), lambda i: (0, 0)),
            pl.BlockSpec((_R, D), lambda i: (i, 0)),
        ],
        out_specs=pl.BlockSpec((_R, D), lambda i: (i, 0)),
        out_shape=jax.ShapeDtypeStruct((N, D), jnp.float32),
    )(parts, W, b.reshape(1, D), x0)


def _tc_last_kernel(p_ref, w_ref, b_ref, x0_ref, wo_ref, bo_ref, o_ref):
    a = p_ref[0] + p_ref[1]
    hh = jnp.dot(a, w_ref[...], preferred_element_type=jnp.float32)
    hh = jnp.maximum(hh + b_ref[...] + x0_ref[...], 0.0)
    z = jnp.dot(hh, wo_ref[...], preferred_element_type=jnp.float32)
    z = z + bo_ref[...]
    m = jnp.max(z, axis=1, keepdims=True)
    lse = jnp.log(jnp.sum(jnp.exp(z - m), axis=1, keepdims=True))
    o_ref[...] = z - m - lse


def _tc_last(parts, W, b, x0, Wout, bout):
    dout = Wout.shape[1]
    return pl.pallas_call(
        _tc_last_kernel,
        grid=(N // _R,),
        in_specs=[
            pl.BlockSpec((NC, _R, D), lambda i: (0, i, 0)),
            pl.BlockSpec((D, D), lambda i: (0, 0)),
            pl.BlockSpec((1, D), lambda i: (0, 0)),
            pl.BlockSpec((_R, D), lambda i: (i, 0)),
            pl.BlockSpec((D, dout), lambda i: (0, 0)),
            pl.BlockSpec((1, dout), lambda i: (0, 0)),
        ],
        out_specs=pl.BlockSpec((_R, dout), lambda i: (i, 0)),
        out_shape=jax.ShapeDtypeStruct((N, dout), jnp.float32),
    )(parts, W, b.reshape(1, D), x0, Wout, bout.reshape(1, dout))


def kernel(x, edge_index, norm, W0, b0, Wc, bc, Wout, bout):
    spread = (jnp.arange(PAD, dtype=jnp.int32) * 911) % N
    pad_blk = jnp.broadcast_to(spread, (NW, PAD))
    src = jnp.concatenate(
        [edge_index[0].reshape(NW, EPW0), pad_blk], axis=1).reshape(-1)
    dst = jnp.concatenate(
        [edge_index[1].reshape(NW, EPW0), pad_blk], axis=1).reshape(-1)
    norm_p = jnp.concatenate(
        [norm.reshape(NW, EPW0),
         jnp.zeros((NW, PAD), jnp.float32)], axis=1).reshape(-1)
    h = _tc_in(x, W0, b0)
    x0 = h
    L = Wc.shape[0]
    for i in range(L - 1):
        parts = _sc_aggregate(h, src, dst, norm_p)
        h = _tc_layer(parts, Wc[i], bc[i], x0)
    parts = _sc_aggregate(h, src, dst, norm_p)
    return _tc_last(parts, Wc[L - 1], bc[L - 1], x0, Wout, bout)


# speedup vs baseline: 1.0409x; 1.0141x over previous
import functools

import jax
import jax.numpy as jnp
from jax import lax
from jax.experimental import pallas as pl
from jax.experimental.pallas import tpu as pltpu
from jax.experimental.pallas import tpu_sc as plsc

N = 10000
E = 320000
D = 128

NC = 2
NS = 16
NW = NC * NS

EPW0 = E // NW
K = 112
PAD = 80
EPW = EPW0 + PAD
CH = EPW // K
NP = 10240
RPT = NP // NS


def _sc_aggregate(h, src, dst, norm):
    mesh = plsc.VectorSubcoreMesh(core_axis_name="c", subcore_axis_name="s",
                                  num_cores=NC)

    @functools.partial(
        pl.kernel, mesh=mesh,
        out_type=jax.ShapeDtypeStruct((NC, NP, D), jnp.float32),
        scratch_types=[
            pltpu.VMEM((K,), jnp.int32),
            pltpu.VMEM((K,), jnp.int32),
            pltpu.VMEM((K,), jnp.int32),
            pltpu.VMEM((K,), jnp.int32),
            pltpu.VMEM((K,), jnp.int32),
            pltpu.VMEM((K,), jnp.int32),
            pltpu.VMEM((K,), jnp.float32),
            pltpu.VMEM((K,), jnp.float32),
            pltpu.VMEM((K,), jnp.float32),
            pltpu.VMEM((K, D), jnp.float32),
            pltpu.VMEM((K, D), jnp.float32),
            pltpu.VMEM((K, D), jnp.float32),
            pltpu.VMEM_SHARED((NP, D), jnp.float32),
            pltpu.SemaphoreType.DMA,
            pltpu.SemaphoreType.DMA,
            pltpu.SemaphoreType.DMA,
            pltpu.SemaphoreType.DMA,
            pltpu.SemaphoreType.DMA,
            pltpu.SemaphoreType.DMA,
            pltpu.SemaphoreType.DMA,
            pltpu.SemaphoreType.DMA,
            pltpu.SemaphoreType.DMA,
            pltpu.SemaphoreType.DMA,
            pltpu.SemaphoreType.DMA,
            pltpu.SemaphoreType.DMA,
            pltpu.SemaphoreType.DMA,
            pltpu.SemaphoreType.DMA,
            pltpu.SemaphoreType.DMA,
        ],
    )
    def agg_kernel(h_hbm, src_hbm, dst_hbm, norm_hbm, out_hbm,
                   src0, src1, src2, dst0, dst1, dst2,
                   norm0, norm1, norm2, rows0, rows1, rows2, agg_sh,
                   ssem0, ssem1, ssem2, dsem0, dsem1, dsem2,
                   nsem0, nsem1, nsem2,
                   gsem0, gsem1, gsem2, csem0, csem1, csem2):
        c = lax.axis_index("c")
        s = lax.axis_index("s")
        wid = s * NC + c
        e0 = wid * EPW

        pltpu.async_copy(src_hbm.at[pl.ds(e0, K)], src0, ssem0)
        pltpu.async_copy(src_hbm.at[pl.ds(e0 + K, K)], src1, ssem1)
        pltpu.async_copy(src_hbm.at[pl.ds(e0 + 2 * K, K)], src2, ssem2)
        pltpu.async_copy(dst_hbm.at[pl.ds(e0, K)], dst0, dsem0)
        pltpu.async_copy(dst_hbm.at[pl.ds(e0 + K, K)], dst1, dsem1)
        pltpu.async_copy(dst_hbm.at[pl.ds(e0 + 2 * K, K)], dst2, dsem2)
        pltpu.async_copy(norm_hbm.at[pl.ds(e0, K)], norm0, nsem0)
        pltpu.async_copy(norm_hbm.at[pl.ds(e0 + K, K)], norm1, nsem1)
        pltpu.async_copy(norm_hbm.at[pl.ds(e0 + 2 * K, K)], norm2, nsem2)

        zero16 = jnp.zeros((16,), jnp.float32)

        def zrow(r, _):
            for j in range(D // 16):
                rows0[r, pl.ds(j * 16, 16)] = zero16
            return 0

        lax.fori_loop(0, 80, zrow, 0)

        def zcopy(t, _):
            pltpu.sync_copy(rows0.at[pl.ds(0, 80)],
                            agg_sh.at[pl.ds(s * RPT + t * 80, 80)])
            return 0

        lax.fori_loop(0, RPT // 80, zcopy, 0)
        plsc.subcore_barrier()

        pltpu.make_async_copy(src_hbm.at[pl.ds(e0, K)], src0, ssem0).wait()
        pltpu.async_copy(h_hbm.at[src0], rows0, gsem0)

        bufs = ((src0, ssem0, dst0, dsem0, norm0, nsem0, rows0, gsem0, csem0),
                (src1, ssem1, dst1, dsem1, norm1, nsem1, rows1, gsem1, csem1),
                (src2, ssem2, dst2, dsem2, norm2, nsem2, rows2, gsem2, csem2))

        def triple(t, _):
            for r in range(3):
                i = 3 * t + r
                (src_c, ssem_c, dst_c, dsem_c, norm_c, nsem_c,
                 rows_c, gsem_c, csem_c) = bufs[r]
                (src_n, ssem_n, dst_n, dsem_n, norm_n, nsem_n,
                 rows_n, gsem_n, csem_n) = bufs[(r + 1) % 3]
                (src_n2, ssem_n2, dst_n2, dsem_n2, norm_n2, nsem_n2,
                 rows_n2, gsem_n2, csem_n2) = bufs[(r + 2) % 3]

                @pl.when(i + 1 < CH)
                def _():
                    pltpu.make_async_copy(
                        src_hbm.at[pl.ds(e0 + (i + 1) * K, K)],
                        src_n, ssem_n).wait()
                    pltpu.async_copy(h_hbm.at[src_n], rows_n, gsem_n)

                @pl.when(i < CH)
                def _():
                    pltpu.make_async_copy(
                        norm_hbm.at[pl.ds(e0 + i * K, K)],
                        norm_c, nsem_c).wait()
                    pltpu.make_async_copy(
                        h_hbm.at[src_c], rows_c, gsem_c).wait()

                    def edge16(g, _):
                        nv = norm_c[pl.ds(g * 16, 16)]
                        for l in range(16):
                            k = g * 16 + l
                            nk = nv[l]
                            for j in range(D // 16):
                                sl = pl.ds(j * 16, 16)
                                rows_c[k, sl] = rows_c[k, sl] * nk
                        return 0

                    lax.fori_loop(0, K // 16, edge16, 0, unroll=2)
                    pltpu.make_async_copy(
                        dst_hbm.at[pl.ds(e0 + i * K, K)],
                        dst_c, dsem_c).wait()
                    pltpu.async_copy(rows_c, agg_sh.at[dst_c], csem_c,
                                     add=True)

                @pl.when(jnp.logical_and(i >= 1, i + 2 < CH))
                def _():
                    pltpu.make_async_copy(
                        rows_n2, agg_sh.at[dst_n2], csem_n2).wait()
                    pltpu.async_copy(
                        src_hbm.at[pl.ds(e0 + (i + 2) * K, K)],
                        src_n2, ssem_n2)
                    pltpu.async_copy(
                        dst_hbm.at[pl.ds(e0 + (i + 2) * K, K)],
                        dst_n2, dsem_n2)
                    pltpu.async_copy(
                        norm_hbm.at[pl.ds(e0 + (i + 2) * K, K)],
                        norm_n2, nsem_n2)

            return 0

        lax.fori_loop(0, (CH + 2) // 3, triple, 0)

        pltpu.make_async_copy(rows0, agg_sh.at[dst0], csem0).wait()
        pltpu.make_async_copy(rows1, agg_sh.at[dst1], csem1).wait()
        pltpu.make_async_copy(rows2, agg_sh.at[dst2], csem2).wait()
        plsc.subcore_barrier()

        pltpu.sync_copy(agg_sh.at[pl.ds(s * RPT, RPT)],
                        out_hbm.at[c, pl.ds(s * RPT, RPT)])

    return agg_kernel(h, src, dst, norm)


_R = 1000


def _tc_in_kernel(x_ref, w_ref, b_ref, o_ref):
    z = jnp.dot(x_ref[...], w_ref[...], preferred_element_type=jnp.float32)
    o_ref[...] = jnp.maximum(z + b_ref[...], 0.0)


def _tc_in(x, W0, b0):
    return pl.pallas_call(
        _tc_in_kernel,
        grid=(N // _R,),
        in_specs=[
            pl.BlockSpec((_R, D), lambda i: (i, 0)),
            pl.BlockSpec((D, D), lambda i: (0, 0)),
            pl.BlockSpec((1, D), lambda i: (0, 0)),
        ],
        out_specs=pl.BlockSpec((_R, D), lambda i: (i, 0)),
        out_shape=jax.ShapeDtypeStruct((N, D), jnp.float32),
    )(x, W0, b0.reshape(1, D))


def _tc_layer_kernel(p_ref, w_ref, b_ref, x0_ref, o_ref):
    a = p_ref[0] + p_ref[1]
    z = jnp.dot(a, w_ref[...], preferred_element_type=jnp.float32)
    o_ref[...] = jnp.maximum(z + b_ref[...] + x0_ref[...], 0.0)


def _tc_layer(parts, W, b, x0):
    return pl.pallas_call(
        _tc_layer_kernel,
        grid=(N // _R,),
        in_specs=[
            pl.BlockSpec((NC, _R, D), lambda i: (0, i, 0)),
            pl.BlockSpec((D, D), lambda i: (0, 0)),
            pl.BlockSpec((1, D), lambda i: (0, 0)),
            pl.BlockSpec((_R, D), lambda i: (i, 0)),
        ],
        out_specs=pl.BlockSpec((_R, D), lambda i: (i, 0)),
        out_shape=jax.ShapeDtypeStruct((N, D), jnp.float32),
    )(parts, W, b.reshape(1, D), x0)


def _tc_last_kernel(p_ref, w_ref, b_ref, x0_ref, wo_ref, bo_ref, o_ref):
    a = p_ref[0] + p_ref[1]
    hh = jnp.dot(a, w_ref[...], preferred_element_type=jnp.float32)
    hh = jnp.maximum(hh + b_ref[...] + x0_ref[...], 0.0)
    z = jnp.dot(hh, wo_ref[...], preferred_element_type=jnp.float32)
    z = z + bo_ref[...]
    m = jnp.max(z, axis=1, keepdims=True)
    lse = jnp.log(jnp.sum(jnp.exp(z - m), axis=1, keepdims=True))
    o_ref[...] = z - m - lse


def _tc_last(parts, W, b, x0, Wout, bout):
    dout = Wout.shape[1]
    return pl.pallas_call(
        _tc_last_kernel,
        grid=(N // _R,),
        in_specs=[
            pl.BlockSpec((NC, _R, D), lambda i: (0, i, 0)),
            pl.BlockSpec((D, D), lambda i: (0, 0)),
            pl.BlockSpec((1, D), lambda i: (0, 0)),
            pl.BlockSpec((_R, D), lambda i: (i, 0)),
            pl.BlockSpec((D, dout), lambda i: (0, 0)),
            pl.BlockSpec((1, dout), lambda i: (0, 0)),
        ],
        out_specs=pl.BlockSpec((_R, dout), lambda i: (i, 0)),
        out_shape=jax.ShapeDtypeStruct((N, dout), jnp.float32),
    )(parts, W, b.reshape(1, D), x0, Wout, bout.reshape(1, dout))


def kernel(x, edge_index, norm, W0, b0, Wc, bc, Wout, bout):
    spread = (jnp.arange(PAD, dtype=jnp.int32) * 911) % N
    pad_blk = jnp.broadcast_to(spread, (NW, PAD))
    src = jnp.concatenate(
        [edge_index[0].reshape(NW, EPW0), pad_blk], axis=1).reshape(-1)
    dst = jnp.concatenate(
        [edge_index[1].reshape(NW, EPW0), pad_blk], axis=1).reshape(-1)
    norm_p = jnp.concatenate(
        [norm.reshape(NW, EPW0),
         jnp.zeros((NW, PAD), jnp.float32)], axis=1).reshape(-1)
    h = _tc_in(x, W0, b0)
    x0 = h
    L = Wc.shape[0]
    for i in range(L - 1):
        parts = _sc_aggregate(h, src, dst, norm_p)
        h = _tc_layer(parts, Wc[i], bc[i], x0)
    parts = _sc_aggregate(h, src, dst, norm_p)
    return _tc_last(parts, Wc[L - 1], bc[L - 1], x0, Wout, bout)
